# Initial kernel scaffold; baseline (speedup 1.0000x reference)
import functools

import jax
import jax.numpy as jnp
from jax.experimental import pallas as pl
from jax.experimental.pallas import tpu as pltpu

M_BLK = 256
N = 64
HID = 32


def _ncm_block_kernel(at_ref, z_ref, order_ref, do_ref, u_ref,
                      w1t_ref, w1tz_ref, b1x_ref, w2x_ref, b2_ref,
                      out_ref):
    m_blk = at_ref.shape[0]
    n = N
    order = order_ref[...]
    do_b = do_ref[...]
    u_b = u_ref[...]

    iota_k = jax.lax.broadcasted_iota(jnp.int32, (m_blk, n, n), 2)
    onehot = (order[:, :, None] == iota_k).astype(jnp.float32)
    g = jax.lax.dot_general(
        onehot, at_ref[...],
        dimension_numbers=(((2,), (1,)), ((0,), (0,))),
        preferred_element_type=jnp.float32)
    g2 = jax.lax.transpose(g, (1, 0, 2))

    lane64 = jax.lax.broadcasted_iota(jnp.int32, (m_blk, n), 1)
    lane2048 = jax.lax.broadcasted_iota(jnp.int32, (m_blk, n * HID), 1)
    kid2048 = lane2048 // HID

    b2_b = jnp.broadcast_to(b2_ref[...], (m_blk, n))
    z_b = z_ref[...]

    outputs = jnp.where(lane64 == do_b, u_b, jnp.zeros((m_blk, n), jnp.float32))

    for i in range(n):
        nodes = order[:, i:i + 1]
        oh64 = lane64 == nodes
        not_int = nodes != do_b
        zg = jnp.sum(jnp.where(oh64, z_b, 0.0), axis=1, keepdims=True)
        b2g = jnp.sum(jnp.where(oh64, b2_b, 0.0), axis=1, keepdims=True)
        masks = g2[i]
        ins = masks * outputs
        h = jax.lax.dot_general(
            ins, w1t_ref[...],
            dimension_numbers=(((1,), (0,)), ((), ())),
            preferred_element_type=jnp.float32)
        h = h + zg * w1tz_ref[...] + b1x_ref[...]
        h = jnp.where(h >= 0, h, 0.01 * h)
        oh2048 = kid2048 == nodes
        val = jnp.sum(jnp.where(oh2048, h * w2x_ref[...], 0.0),
                      axis=1, keepdims=True) + b2g
        outputs = jnp.where(oh64 & not_int, val, outputs)

    out_ref[...] = outputs


def kernel(Z, A, order, do, W1, b1, W2, b2):
    m, n = Z.shape
    hid = W1.shape[1]
    u = 2.0 + jax.random.normal(jax.random.key(42), (m,), dtype=Z.dtype)

    At = jnp.swapaxes(A, 1, 2)
    w1t = jnp.transpose(W1, (2, 0, 1)).reshape(n + 1, n * hid)
    w1tj = w1t[:n]
    w1tz = w1t[n:]
    b1x = b1.reshape(1, n * hid)
    w2x = W2.reshape(1, n * hid)
    b2r = b2.reshape(1, n)
    do2 = do.reshape(m, 1)
    u2 = u.reshape(m, 1)

    grid = m // M_BLK
    out = pl.pallas_call(
        _ncm_block_kernel,
        grid=(grid,),
        in_specs=[
            pl.BlockSpec((M_BLK, n, n), lambda b: (b, 0, 0)),
            pl.BlockSpec((M_BLK, n), lambda b: (b, 0)),
            pl.BlockSpec((M_BLK, n), lambda b: (b, 0)),
            pl.BlockSpec((M_BLK, 1), lambda b: (b, 0)),
            pl.BlockSpec((M_BLK, 1), lambda b: (b, 0)),
            pl.BlockSpec((n, n * hid), lambda b: (0, 0)),
            pl.BlockSpec((1, n * hid), lambda b: (0, 0)),
            pl.BlockSpec((1, n * hid), lambda b: (0, 0)),
            pl.BlockSpec((1, n * hid), lambda b: (0, 0)),
            pl.BlockSpec((1, n), lambda b: (0, 0)),
        ],
        out_specs=pl.BlockSpec((M_BLK, n), lambda b: (b, 0)),
        out_shape=jax.ShapeDtypeStruct((m, n), Z.dtype),
    )(At, Z, order, do2, u2, w1tj, w1tz, b1x, w2x, b2r)
    return out


# speedup vs baseline: 9.8085x; 9.8085x over previous
import jax
import jax.numpy as jnp
from jax.experimental import pallas as pl
from jax.experimental.pallas import tpu as pltpu

M_BLK = 256
N = 64
HID = 32


def _ncm_block_kernel(atext_ref, order_ref, do_ref, u_ref,
                      w1t_ref, b1x_ref, w2x_ref, b2_ref,
                      out_ref, gx_ref, ohm_ref):
    m_blk = atext_ref.shape[0]
    n = N
    order = order_ref[...]
    do_b = do_ref[...]
    u_b = u_ref[...]

    iota_k = jax.lax.broadcasted_iota(jnp.int32, (m_blk, n, n), 2)
    onehot = (order[:, :, None] == iota_k).astype(jnp.float32)
    gx = jax.lax.dot_general(
        onehot, atext_ref[...],
        dimension_numbers=(((2,), (1,)), ((0,), (0,))),
        preferred_element_type=jnp.float32)
    gx_ref[...] = jax.lax.transpose(gx, (1, 0, 2))

    not_int = (order != do_b).astype(jnp.float32)
    ohm = onehot * not_int[:, :, None]
    ohm_ref[:, :, :n] = jax.lax.transpose(ohm, (1, 0, 2))
    ohm_ref[:, :, n:] = jnp.zeros((n, m_blk, 1), jnp.float32)

    lane65 = jax.lax.broadcasted_iota(jnp.int32, (m_blk, n + 1), 1)
    b2_b = jnp.broadcast_to(b2_ref[...], (m_blk, n))

    outputs0 = jnp.where(lane65 == do_b, u_b,
                         jnp.where(lane65 == n, 1.0, 0.0))

    def body(i, outputs):
        ge = gx_ref[i]
        ohm_i = ohm_ref[i]
        ins = ge * outputs
        h = jax.lax.dot_general(
            ins, w1t_ref[...],
            dimension_numbers=(((1,), (0,)), ((), ())),
            preferred_element_type=jnp.float32)
        h = h + b1x_ref[...]
        h = jnp.where(h >= 0, h, 0.01 * h)
        hw = h * w2x_ref[...]
        hw = hw[:, :1024] + hw[:, 1024:]
        hw = hw[:, :512] + hw[:, 512:]
        hw = hw[:, :256] + hw[:, 256:]
        hw = hw[:, :128] + hw[:, 128:]
        s = hw[:, :64] + hw[:, 64:]
        val = jnp.sum(ohm_i[:, :n] * (s + b2_b), axis=1, keepdims=True)
        return jnp.where(ohm_i != 0.0, val, outputs)

    outputs = jax.lax.fori_loop(0, n, body, outputs0)
    out_ref[...] = outputs[:, :n]


def kernel(Z, A, order, do, W1, b1, W2, b2):
    m, n = Z.shape
    hid = W1.shape[1]
    u = 2.0 + jax.random.normal(jax.random.key(42), (m,), dtype=Z.dtype)

    at_ext = jnp.concatenate([jnp.swapaxes(A, 1, 2), Z[:, :, None]], axis=2)
    w1t = jnp.transpose(W1, (2, 1, 0)).reshape(n + 1, hid * n)
    b1x = b1.T.reshape(1, hid * n)
    w2x = W2.T.reshape(1, hid * n)
    b2r = b2.reshape(1, n)
    do2 = do.reshape(m, 1)
    u2 = u.reshape(m, 1)

    grid = m // M_BLK
    out = pl.pallas_call(
        _ncm_block_kernel,
        grid=(grid,),
        in_specs=[
            pl.BlockSpec((M_BLK, n, n + 1), lambda b: (b, 0, 0)),
            pl.BlockSpec((M_BLK, n), lambda b: (b, 0)),
            pl.BlockSpec((M_BLK, 1), lambda b: (b, 0)),
            pl.BlockSpec((M_BLK, 1), lambda b: (b, 0)),
            pl.BlockSpec((n + 1, hid * n), lambda b: (0, 0)),
            pl.BlockSpec((1, hid * n), lambda b: (0, 0)),
            pl.BlockSpec((1, hid * n), lambda b: (0, 0)),
            pl.BlockSpec((1, n), lambda b: (0, 0)),
        ],
        out_specs=pl.BlockSpec((M_BLK, n), lambda b: (b, 0)),
        out_shape=jax.ShapeDtypeStruct((m, n), Z.dtype),
        scratch_shapes=[
            pltpu.VMEM((n, M_BLK, n + 1), jnp.float32),
            pltpu.VMEM((n, M_BLK, n + 1), jnp.float32),
        ],
    )(at_ext, order, do2, u2, w1t, b1x, w2x, b2r)
    return out


# speedup vs baseline: 12.1876x; 1.2426x over previous
import jax
import jax.numpy as jnp
from jax.experimental import pallas as pl
from jax.experimental.pallas import tpu as pltpu

M_BLK = 256
N = 64
HID = 32
NL = N + 2


def _ncm_block_kernel(atext_ref, order_ref, do_ref, u_ref,
                      w1t2_ref, w2pos_ref, b2_ref,
                      out_ref, gx_ref):
    m_blk = atext_ref.shape[0]
    n = N
    order = order_ref[...]
    do_b = do_ref[...]
    u_b = u_ref[...]

    iota_k = jax.lax.broadcasted_iota(jnp.int32, (m_blk, n, n), 2)
    onehot = (order[:, :, None] == iota_k).astype(jnp.bfloat16)
    gx = jax.lax.dot_general(
        onehot, atext_ref[...],
        dimension_numbers=(((2,), (1,)), ((0,), (0,))),
        preferred_element_type=jnp.float32)
    gx_ref[:, :, :NL] = jax.lax.transpose(gx.astype(jnp.bfloat16), (1, 0, 2))
    code = jnp.where(order != do_b, order + 1, 0).astype(jnp.bfloat16)
    gx_ref[:, :, NL] = jax.lax.transpose(code, (1, 0))

    lane = jax.lax.broadcasted_iota(jnp.int32, (m_blk, NL), 1)
    b2_b = jnp.broadcast_to(b2_ref[...], (m_blk, n))
    w2pos = w2pos_ref[...] != 0

    outputs0 = jnp.where(lane == do_b, u_b,
                         jnp.where(lane >= n, 1.0, 0.0))

    def body(i, outputs):
        gef = gx_ref[i]
        ge = gef[:, :NL].astype(jnp.float32)
        cmb = gef[:, NL:].astype(jnp.int32)
        ins = (ge * outputs).astype(jnp.bfloat16)
        t = jax.lax.dot_general(
            ins, w1t2_ref[...],
            dimension_numbers=(((1,), (0,)), ((), ())),
            preferred_element_type=jnp.float32)
        t = t.astype(jnp.bfloat16)
        q = t * jnp.bfloat16(0.01)
        hw = jnp.where(w2pos, jnp.maximum(t, q), jnp.minimum(t, q))
        hw = hw[:, :1024] + hw[:, 1024:]
        hw = hw[:, :512] + hw[:, 512:]
        hw = hw[:, :256] + hw[:, 256:]
        hw = hw[:, :128] + hw[:, 128:]
        s = (hw[:, :64] + hw[:, 64:]).astype(jnp.float32)
        oh = lane + 1 == cmb
        val = jnp.sum(jnp.where(oh[:, :n], s + b2_b, 0.0),
                      axis=1, keepdims=True)
        return jnp.where(oh, val, outputs)

    outputs = jax.lax.fori_loop(0, n, body, outputs0)
    out_ref[...] = outputs[:, :n]


def kernel(Z, A, order, do, W1, b1, W2, b2):
    m, n = Z.shape
    hid = W1.shape[1]
    u = 2.0 + jax.random.normal(jax.random.key(42), (m,), dtype=Z.dtype)

    at_ext = jnp.concatenate(
        [jnp.swapaxes(A, 1, 2), Z[:, :, None], jnp.ones((m, n, 1), Z.dtype)],
        axis=2).astype(jnp.bfloat16)
    w1f = jnp.transpose(W1, (2, 1, 0)) * W2.T[None]
    bias = (b1.T * W2.T)[None]
    w1t2 = jnp.concatenate([w1f, bias], 0).reshape(n + 2, hid * n)
    w1t2 = w1t2.astype(jnp.bfloat16)
    w2pos = (W2.T.reshape(1, hid * n) > 0).astype(jnp.float32)
    b2r = b2.reshape(1, n)
    do2 = do.reshape(m, 1)
    u2 = u.reshape(m, 1)

    grid = m // M_BLK
    out = pl.pallas_call(
        _ncm_block_kernel,
        grid=(grid,),
        in_specs=[
            pl.BlockSpec((M_BLK, n, NL), lambda b: (b, 0, 0)),
            pl.BlockSpec((M_BLK, n), lambda b: (b, 0)),
            pl.BlockSpec((M_BLK, 1), lambda b: (b, 0)),
            pl.BlockSpec((M_BLK, 1), lambda b: (b, 0)),
            pl.BlockSpec((NL, hid * n), lambda b: (0, 0)),
            pl.BlockSpec((1, hid * n), lambda b: (0, 0)),
            pl.BlockSpec((1, n), lambda b: (0, 0)),
        ],
        out_specs=pl.BlockSpec((M_BLK, n), lambda b: (b, 0)),
        out_shape=jax.ShapeDtypeStruct((m, n), Z.dtype),
        scratch_shapes=[
            pltpu.VMEM((n, M_BLK, NL + 1), jnp.bfloat16),
        ],
    )(at_ext, order, do2, u2, w1t2, w2pos, b2r)
    return out
